# baseline (device time: 170340 ns/iter reference)
import jax
import jax.numpy as jnp
from jax import lax
from jax.experimental import pallas as pl
from jax.experimental.pallas import tpu as pltpu

N_DEV = 4
SQ_PER = 256
SQ = N_DEV * SQ_PER
SKV = 4096
H_PER = 8
DH = 128
D_MODEL = 1024
SCALE = 0.08838834764831843
QBLK = 512
N_QBLK = SQ // QBLK

_MESH = pl.DeviceIdType.MESH


def _mm(a, b, dims):
    return lax.dot_general(a, b, (dims, ((), ())),
                           preferred_element_type=jnp.float32)


def kernel(x, Wq, K_ext, V_ext, Wo):
    my = lax.axis_index("i")

    x2 = x[0].astype(jnp.bfloat16)
    wq = Wq.astype(jnp.bfloat16)
    k = lax.dynamic_slice_in_dim(K_ext[0], my * H_PER, H_PER, axis=1)
    v = lax.dynamic_slice_in_dim(V_ext[0], my * H_PER, H_PER, axis=1)
    k = jnp.transpose(k, (1, 0, 2)).astype(jnp.bfloat16)
    v = jnp.transpose(v, (1, 0, 2)).astype(jnp.bfloat16)
    wo = Wo.astype(jnp.bfloat16)

    def body(x_ref, wq_ref, k_ref, v_ref, wo_ref, out_ref,
             xg_ref, ps_ref, pr_ref,
             xs_sems, xr_sems, ps_sems, pr_sems):
        my_pos = lax.axis_index("i")

        bar = pltpu.get_barrier_semaphore()
        for d in range(1, N_DEV):
            peer = lax.rem(my_pos + d, N_DEV)
            pl.semaphore_signal(bar, inc=1, device_id=(peer,),
                                device_id_type=_MESH)
        pl.semaphore_wait(bar, N_DEV - 1)

        xg_ref[pl.ds(my_pos, 1)] = x_ref[...].reshape(1, SQ_PER, D_MODEL)
        x_sends = []
        for d in range(1, N_DEV):
            peer = lax.rem(my_pos + d, N_DEV)
            rdma = pltpu.make_async_remote_copy(
                src_ref=xg_ref.at[pl.ds(my_pos, 1)],
                dst_ref=xg_ref.at[pl.ds(my_pos, 1)],
                send_sem=xs_sems.at[d - 1],
                recv_sem=xr_sems.at[my_pos],
                device_id=(peer,),
                device_id_type=_MESH,
            )
            rdma.start()
            x_sends.append(rdma)
        for d in range(1, N_DEV):
            j = lax.rem(my_pos + d, N_DEV)
            recv = pltpu.make_async_remote_copy(
                src_ref=xg_ref.at[pl.ds(j, 1)],
                dst_ref=xg_ref.at[pl.ds(j, 1)],
                send_sem=xs_sems.at[d - 1],
                recv_sem=xr_sems.at[j],
                device_id=(j,),
                device_id_type=_MESH,
            )
            recv.wait_recv()

        x_full = xg_ref[...].reshape(SQ, D_MODEL)
        q = _mm(x_full, wq_ref[...], (((1,), (0,))))
        q = q.astype(jnp.bfloat16).reshape(SQ, H_PER, DH)

        for qb in range(N_QBLK):
            q0 = qb * QBLK
            qi = lax.broadcasted_iota(jnp.int32, (QBLK, SKV), 0) + q0
            ki = lax.broadcasted_iota(jnp.int32, (QBLK, SKV), 1)
            mask = (jnp.abs(qi - ki) <= 128) | (ki < 32) | (qi < 32)
            acc = jnp.zeros((QBLK, D_MODEL), jnp.float32)
            for h in range(H_PER):
                qh = q[q0:q0 + QBLK, h, :]
                s = _mm(qh, k_ref[h], (((1,), (1,)))) * SCALE
                s = jnp.where(mask, s, -1e9)
                m = jnp.max(s, axis=1, keepdims=True)
                w = jnp.exp(s - m)
                w = (w / jnp.sum(w, axis=1, keepdims=True)).astype(jnp.bfloat16)
                ctx = _mm(w, v_ref[h], (((1,), (0,)))).astype(jnp.bfloat16)
                acc = acc + _mm(ctx, wo_ref[h * DH:(h + 1) * DH, :],
                                (((1,), (0,))))
            ps_ref[pl.ds(qb * (QBLK // SQ_PER), QBLK // SQ_PER)] = (
                acc.reshape(QBLK // SQ_PER, SQ_PER, D_MODEL)
                .astype(jnp.bfloat16))

        pr_ref[pl.ds(my_pos, 1)] = ps_ref[pl.ds(my_pos, 1)]
        p_sends = []
        for d in range(1, N_DEV):
            peer = lax.rem(my_pos + d, N_DEV)
            rdma = pltpu.make_async_remote_copy(
                src_ref=ps_ref.at[pl.ds(peer, 1)],
                dst_ref=pr_ref.at[pl.ds(my_pos, 1)],
                send_sem=ps_sems.at[d - 1],
                recv_sem=pr_sems.at[my_pos],
                device_id=(peer,),
                device_id_type=_MESH,
            )
            rdma.start()
            p_sends.append(rdma)
        for d in range(1, N_DEV):
            j = lax.rem(my_pos + d, N_DEV)
            recv = pltpu.make_async_remote_copy(
                src_ref=ps_ref.at[pl.ds(j, 1)],
                dst_ref=pr_ref.at[pl.ds(j, 1)],
                send_sem=ps_sems.at[d - 1],
                recv_sem=pr_sems.at[j],
                device_id=(j,),
                device_id_type=_MESH,
            )
            recv.wait_recv()

        out_ref[...] = jnp.sum(pr_ref[...].astype(jnp.float32), axis=0)

        for rdma in x_sends + p_sends:
            rdma.wait_send()

    out = pl.pallas_call(
        body,
        out_shape=jax.ShapeDtypeStruct((SQ_PER, D_MODEL), jnp.float32),
        in_specs=[pl.BlockSpec(memory_space=pltpu.VMEM)] * 5,
        out_specs=pl.BlockSpec(memory_space=pltpu.VMEM),
        scratch_shapes=[
            pltpu.VMEM((N_DEV, SQ_PER, D_MODEL), jnp.bfloat16),
            pltpu.VMEM((N_DEV, SQ_PER, D_MODEL), jnp.bfloat16),
            pltpu.VMEM((N_DEV, SQ_PER, D_MODEL), jnp.bfloat16),
            pltpu.SemaphoreType.DMA((N_DEV - 1,)),
            pltpu.SemaphoreType.DMA((N_DEV,)),
            pltpu.SemaphoreType.DMA((N_DEV - 1,)),
            pltpu.SemaphoreType.DMA((N_DEV,)),
        ],
        compiler_params=pltpu.CompilerParams(collective_id=0),
    )(x2, wq, k, v, wo)
    return out.reshape(1, SQ_PER, D_MODEL)


# device time: 98583 ns/iter; 1.7279x vs baseline; 1.7279x over previous
import jax
import jax.numpy as jnp
from jax import lax
from jax.experimental import pallas as pl
from jax.experimental.pallas import tpu as pltpu

N_DEV = 4
SQ_PER = 256
SQ = N_DEV * SQ_PER
SKV = 4096
H_PER = 8
DH = 128
D_MODEL = 1024
SCALE = 0.08838834764831843
GW = 128
WW = 512
NGLOB = 32

_MESH = pl.DeviceIdType.MESH


def _mm(a, b, dims):
    return lax.dot_general(a, b, (dims, ((), ())),
                           preferred_element_type=jnp.float32)


def kernel(x, Wq, K_ext, V_ext, Wo):
    my = lax.axis_index("i")

    x2 = x[0].astype(jnp.bfloat16)
    wq = Wq.astype(jnp.bfloat16)
    k = lax.dynamic_slice_in_dim(K_ext[0], my * H_PER, H_PER, axis=1)
    v = lax.dynamic_slice_in_dim(V_ext[0], my * H_PER, H_PER, axis=1)
    k = jnp.transpose(k, (1, 0, 2)).astype(jnp.bfloat16)
    v = jnp.transpose(v, (1, 0, 2)).astype(jnp.bfloat16)
    wo = Wo.astype(jnp.bfloat16)

    def body(x_ref, wq_ref, k_ref, v_ref, wo_ref, out_ref,
             xg_ref, ps_ref, pr_ref,
             xs_sems, xr_sems, ps_sems, pr_sems):
        my_pos = lax.axis_index("i")

        bar = pltpu.get_barrier_semaphore()
        for d in range(1, N_DEV):
            peer = lax.rem(my_pos + d, N_DEV)
            pl.semaphore_signal(bar, inc=1, device_id=(peer,),
                                device_id_type=_MESH)
        pl.semaphore_wait(bar, N_DEV - 1)

        xg_ref[pl.ds(my_pos, 1)] = x_ref[...].reshape(1, SQ_PER, D_MODEL)
        x_sends = []
        for d in range(1, N_DEV):
            peer = lax.rem(my_pos + d, N_DEV)
            rdma = pltpu.make_async_remote_copy(
                src_ref=xg_ref.at[pl.ds(my_pos, 1)],
                dst_ref=xg_ref.at[pl.ds(my_pos, 1)],
                send_sem=xs_sems.at[d - 1],
                recv_sem=xr_sems.at[my_pos],
                device_id=(peer,),
                device_id_type=_MESH,
            )
            rdma.start()
            x_sends.append(rdma)

        wq_v = wq_ref[...]

        def qproj(xblk, rows):
            qf = _mm(xblk, wq_v, (((1,), (0,))))
            return qf.astype(jnp.bfloat16).reshape(rows, H_PER, DH)

        def attend_block(qb3, q0):
            wstart = pl.multiple_of(jnp.maximum(GW, q0 - 128), 128)
            qi_l = lax.broadcasted_iota(jnp.int32, (SQ_PER, GW), 0) + q0
            ki_l = lax.broadcasted_iota(jnp.int32, (SQ_PER, GW), 1)
            mask_low = ((jnp.abs(qi_l - ki_l) <= 128) | (ki_l < NGLOB)
                        | (qi_l < NGLOB))
            qi_w = lax.broadcasted_iota(jnp.int32, (SQ_PER, WW), 0) + q0
            ki_w = lax.broadcasted_iota(jnp.int32, (SQ_PER, WW), 1) + wstart
            mask_win = (jnp.abs(qi_w - ki_w) <= 128) | (qi_w < NGLOB)

            acc = jnp.zeros((SQ_PER, D_MODEL), jnp.float32)
            for h in range(H_PER):
                qh = qb3[:, h, :]
                k_low = k_ref[h, 0:GW, :]
                k_win = k_ref[h, pl.ds(wstart, WW), :]
                s_low = _mm(qh, k_low, (((1,), (1,)))) * SCALE
                s_win = _mm(qh, k_win, (((1,), (1,)))) * SCALE
                s_low = jnp.where(mask_low, s_low, -1e9)
                s_win = jnp.where(mask_win, s_win, -1e9)
                m = jnp.maximum(jnp.max(s_low, axis=1, keepdims=True),
                                jnp.max(s_win, axis=1, keepdims=True))
                w_low = jnp.exp(s_low - m)
                w_win = jnp.exp(s_win - m)
                den = (jnp.sum(w_low, axis=1, keepdims=True)
                       + jnp.sum(w_win, axis=1, keepdims=True))
                wl = (w_low / den).astype(jnp.bfloat16)
                ww = (w_win / den).astype(jnp.bfloat16)
                v_low = v_ref[h, 0:GW, :]
                v_win = v_ref[h, pl.ds(wstart, WW), :]
                ctx = (_mm(wl, v_low, (((1,), (0,))))
                       + _mm(ww, v_win, (((1,), (0,))))).astype(jnp.bfloat16)
                woh = wo_ref[h * DH:(h + 1) * DH, :]
                acc = acc + _mm(ctx, woh, (((1,), (0,))))
            return acc

        acc_my = attend_block(qproj(x_ref[...], SQ_PER), my_pos * SQ_PER)
        ps_ref[pl.ds(my_pos, 1)] = (
            acc_my.reshape(1, SQ_PER, D_MODEL).astype(jnp.bfloat16))

        for d in range(1, N_DEV):
            j = lax.rem(my_pos + d, N_DEV)
            recv = pltpu.make_async_remote_copy(
                src_ref=xg_ref.at[pl.ds(j, 1)],
                dst_ref=xg_ref.at[pl.ds(j, 1)],
                send_sem=xs_sems.at[d - 1],
                recv_sem=xr_sems.at[j],
                device_id=(j,),
                device_id_type=_MESH,
            )
            recv.wait_recv()

        q32 = qproj(xg_ref[0, 0:NGLOB, :], NGLOB)

        acc_g = jnp.zeros((NGLOB, D_MODEL), jnp.float32)
        for h in range(H_PER):
            qh = q32[:, h, :]
            s = _mm(qh, k_ref[h], (((1,), (1,)))) * SCALE
            m = jnp.max(s, axis=1, keepdims=True)
            w = jnp.exp(s - m)
            w = (w / jnp.sum(w, axis=1, keepdims=True)).astype(jnp.bfloat16)
            ctx = _mm(w, v_ref[h], (((1,), (0,)))).astype(jnp.bfloat16)
            acc_g = acc_g + _mm(ctx, wo_ref[h * DH:(h + 1) * DH, :],
                                (((1,), (0,))))
        acc_g16 = acc_g.astype(jnp.bfloat16)

        @pl.when(my_pos == 0)
        def _():
            ps_ref[0, 0:NGLOB, :] = acc_g16

        p_sends = []
        for d in range(1, N_DEV):
            b = lax.rem(my_pos + d, N_DEV)
            xb = xg_ref[pl.ds(b, 1)].reshape(SQ_PER, D_MODEL)
            acc_b = attend_block(qproj(xb, SQ_PER), b * SQ_PER)
            ps_ref[pl.ds(b, 1)] = (
                acc_b.reshape(1, SQ_PER, D_MODEL).astype(jnp.bfloat16))

            @pl.when(b == 0)
            def _():
                ps_ref[0, 0:NGLOB, :] = acc_g16

            rdma = pltpu.make_async_remote_copy(
                src_ref=ps_ref.at[pl.ds(b, 1)],
                dst_ref=pr_ref.at[pl.ds(my_pos, 1)],
                send_sem=ps_sems.at[d - 1],
                recv_sem=pr_sems.at[my_pos],
                device_id=(b,),
                device_id_type=_MESH,
            )
            rdma.start()
            p_sends.append(rdma)

        pr_ref[pl.ds(my_pos, 1)] = ps_ref[pl.ds(my_pos, 1)]
        for d in range(1, N_DEV):
            j = lax.rem(my_pos + d, N_DEV)
            recv = pltpu.make_async_remote_copy(
                src_ref=ps_ref.at[pl.ds(j, 1)],
                dst_ref=pr_ref.at[pl.ds(j, 1)],
                send_sem=ps_sems.at[d - 1],
                recv_sem=pr_sems.at[j],
                device_id=(j,),
                device_id_type=_MESH,
            )
            recv.wait_recv()

        out_ref[...] = jnp.sum(pr_ref[...].astype(jnp.float32), axis=0)

        for rdma in x_sends + p_sends:
            rdma.wait_send()

    out = pl.pallas_call(
        body,
        out_shape=jax.ShapeDtypeStruct((SQ_PER, D_MODEL), jnp.float32),
        in_specs=[pl.BlockSpec(memory_space=pltpu.VMEM)] * 5,
        out_specs=pl.BlockSpec(memory_space=pltpu.VMEM),
        scratch_shapes=[
            pltpu.VMEM((N_DEV, SQ_PER, D_MODEL), jnp.bfloat16),
            pltpu.VMEM((N_DEV, SQ_PER, D_MODEL), jnp.bfloat16),
            pltpu.VMEM((N_DEV, SQ_PER, D_MODEL), jnp.bfloat16),
            pltpu.SemaphoreType.DMA((N_DEV - 1,)),
            pltpu.SemaphoreType.DMA((N_DEV,)),
            pltpu.SemaphoreType.DMA((N_DEV - 1,)),
            pltpu.SemaphoreType.DMA((N_DEV,)),
        ],
        compiler_params=pltpu.CompilerParams(collective_id=0),
    )(x2, wq, k, v, wo)
    return out.reshape(1, SQ_PER, D_MODEL)
